# Initial kernel scaffold; baseline (speedup 1.0000x reference)
import jax
import jax.numpy as jnp
from jax.experimental import pallas as pl


def kernel(x, edge_index, W1, b1, conv_ws, W2, b2):
    raise NotImplementedError("write your pallas kernel here")


# speedup vs baseline: 7.7207x; 7.7207x over previous
import functools

import jax
import jax.numpy as jnp
from jax import lax
from jax.experimental import pallas as pl
from jax.experimental.pallas import tpu as pltpu
from jax.experimental.pallas import tpu_sc as plsc

ALPHA = 0.1
NC, NS = 2, 16
NW = NC * NS
L = 16
CH = 128
PAD = 16


def _sc_mesh():
    return plsc.VectorSubcoreMesh(
        core_axis_name="c", subcore_axis_name="s", num_cores=NC, num_subcores=NS
    )


def _spans(N):
    base_sz = (N // NS) // 8 * 8
    last_sz = N - base_sz * (NS - 1)
    return base_sz, last_sz


@functools.lru_cache(maxsize=None)
def _make_deg_kernel(nchunk, N):
    base_sz, last_sz = _spans(N)
    tail = last_sz - base_sz
    toff = (NS - 1) * base_sz + base_sz

    @functools.partial(
        pl.kernel,
        mesh=_sc_mesh(),
        out_type=jax.ShapeDtypeStruct((NC * N,), jnp.float32),
        scratch_types=[
            pltpu.VMEM_SHARED((N + PAD,), jnp.float32),
            pltpu.VMEM((nchunk, CH), jnp.int32),
            pltpu.VMEM((CH,), jnp.float32),
            pltpu.VMEM((base_sz + tail,), jnp.float32),
        ],
    )
    def deg_kernel(col_hbm, out_hbm, acc_s, cidx_v, ones_v, zb_v):
        cid = lax.axis_index("c")
        sid = lax.axis_index("s")
        wid = sid * NC + cid
        pltpu.sync_copy(col_hbm.at[wid], cidx_v)

        onesv = jnp.ones((L,), jnp.float32)
        zerov = jnp.zeros((L,), jnp.float32)

        def fill_ones(i, _):
            ones_v[pl.ds(i * L, L)] = onesv
            return 0

        lax.fori_loop(0, CH // L, fill_ones, 0)

        def fill_zero(i, _):
            zb_v[pl.ds(i * L, L)] = zerov
            return 0

        lax.fori_loop(0, (base_sz + tail) // L, fill_zero, 0)

        start = sid * base_sz
        pltpu.sync_copy(zb_v.at[pl.ds(0, base_sz)], acc_s.at[pl.ds(start, base_sz)])

        @pl.when(sid == NS - 1)
        def _():
            pltpu.sync_copy(
                zb_v.at[pl.ds(0, tail + PAD)], acc_s.at[pl.ds(toff, tail + PAD)]
            )

        plsc.subcore_barrier()

        def body(j, _):
            pltpu.sync_copy(ones_v, acc_s.at[cidx_v.at[j]], add=True)
            return 0

        lax.fori_loop(0, nchunk, body, 0)
        plsc.subcore_barrier()

        pltpu.sync_copy(acc_s.at[pl.ds(start, base_sz)], zb_v.at[pl.ds(0, base_sz)])
        pltpu.sync_copy(
            zb_v.at[pl.ds(0, base_sz)],
            out_hbm.at[pl.ds(cid * N + start, base_sz)],
        )

        @pl.when(sid == NS - 1)
        def _():
            pltpu.sync_copy(acc_s.at[pl.ds(toff, tail)], zb_v.at[pl.ds(0, tail)])
            pltpu.sync_copy(
                zb_v.at[pl.ds(0, tail)], out_hbm.at[pl.ds(cid * N + toff, tail)]
            )

    return deg_kernel


@functools.lru_cache(maxsize=None)
def _make_agg_kernel(nchunk, N, D):
    base_sz, last_sz = _spans(N)
    tail = last_sz - base_sz
    toff = (NS - 1) * base_sz + base_sz
    zr = 16

    @functools.partial(
        pl.kernel,
        mesh=_sc_mesh(),
        out_type=jax.ShapeDtypeStruct((NC, N, D), jnp.float32),
        scratch_types=[
            pltpu.VMEM_SHARED((N + PAD, D), jnp.float32),
            pltpu.VMEM((nchunk, CH), jnp.int32),
            pltpu.VMEM((nchunk, CH), jnp.int32),
            pltpu.VMEM((CH, D), jnp.float32),
            pltpu.VMEM((zr, D), jnp.float32),
            pltpu.SemaphoreType.DMA,
        ],
    )
    def agg_kernel(g_hbm, row_hbm, col_hbm, out_hbm, acc_s, ridx_v, cidx_v,
                   rows_v, zb_v, gsem):
        cid = lax.axis_index("c")
        sid = lax.axis_index("s")
        wid = sid * NC + cid
        pltpu.sync_copy(row_hbm.at[wid], ridx_v)
        pltpu.sync_copy(col_hbm.at[wid], cidx_v)

        zerov = jnp.zeros((L,), jnp.float32)

        def fill_zero(i, _):
            r = i // (D // L)
            c = i % (D // L)
            zb_v[r, pl.ds(c * L, L)] = zerov
            return 0

        lax.fori_loop(0, zr * D // L, fill_zero, 0)

        start = sid * base_sz
        nz = jnp.where(sid == NS - 1, (last_sz + PAD) // zr, base_sz // zr)

        def zero_acc(k, _):
            pltpu.sync_copy(zb_v, acc_s.at[pl.ds(start + k * zr, zr)])
            return 0

        lax.fori_loop(0, nz, zero_acc, 0)
        plsc.subcore_barrier()

        def body(j, _):
            pltpu.async_copy(g_hbm.at[ridx_v.at[j]], rows_v, gsem).wait()
            pltpu.sync_copy(rows_v, acc_s.at[cidx_v.at[j]], add=True)
            return 0

        lax.fori_loop(0, nchunk, body, 0)
        plsc.subcore_barrier()

        def wb_body(k, _):
            off = start + k * zr
            pltpu.sync_copy(acc_s.at[pl.ds(off, zr)], zb_v)
            pltpu.sync_copy(zb_v, out_hbm.at[cid, pl.ds(off, zr)])
            return 0

        nw_ = jnp.where(sid == NS - 1, last_sz // zr, base_sz // zr)
        lax.fori_loop(0, nw_, wb_body, 0)

    return agg_kernel


def _dinv_of(degp_blk):
    deg = degp_blk[:, 0] + degp_blk[:, 1]
    return jnp.where(deg > 0.0, lax.rsqrt(deg), 0.0)[:, None]


def _fc1_body(x_ref, w1_ref, b1_ref, degp_ref, h_ref, g_ref):
    h = jnp.maximum(
        jnp.dot(x_ref[...], w1_ref[...], preferred_element_type=jnp.float32)
        + b1_ref[...],
        0.0,
    )
    h_ref[...] = h
    g_ref[...] = h * _dinv_of(degp_ref[...])


def _layer_body(p_ref, x0_ref, degp_ref, w_ref, g_ref):
    dinv = _dinv_of(degp_ref[...])
    p = p_ref[...]
    t = (1.0 - ALPHA) * dinv * (p[0] + p[1]) + ALPHA * x0_ref[...]
    h = jnp.maximum(
        jnp.dot(t, w_ref[...], preferred_element_type=jnp.float32), 0.0
    )
    g_ref[...] = h * dinv


def _final_body(p_ref, x0_ref, degp_ref, w_ref, w2_ref, b2_ref, out_ref):
    dinv = _dinv_of(degp_ref[...])
    p = p_ref[...]
    t = (1.0 - ALPHA) * dinv * (p[0] + p[1]) + ALPHA * x0_ref[...]
    h = jnp.maximum(
        jnp.dot(t, w_ref[...], preferred_element_type=jnp.float32), 0.0
    )
    out_ref[...] = (
        jnp.dot(h, w2_ref[...], preferred_element_type=jnp.float32) + b2_ref[...]
    )


def _row_blk(i):
    return (i, 0)


@functools.lru_cache(maxsize=None)
def _make_tc_kernels(N, D, D_OUT, R):
    grid = (N // R,)
    mat = pl.BlockSpec((D, D), lambda i: (0, 0))
    vec = pl.BlockSpec((1, D), lambda i: (0, 0))
    rows = pl.BlockSpec((R, D), _row_blk)
    degp = pl.BlockSpec((R, NC), _row_blk)
    part = pl.BlockSpec((NC, R, D), lambda i: (0, i, 0))

    fc1 = pl.pallas_call(
        _fc1_body,
        grid=grid,
        in_specs=[rows, mat, vec, degp],
        out_specs=[rows, rows],
        out_shape=[
            jax.ShapeDtypeStruct((N, D), jnp.float32),
            jax.ShapeDtypeStruct((N, D), jnp.float32),
        ],
    )
    layer = pl.pallas_call(
        _layer_body,
        grid=grid,
        in_specs=[part, rows, degp, mat],
        out_specs=rows,
        out_shape=jax.ShapeDtypeStruct((N, D), jnp.float32),
    )
    final = pl.pallas_call(
        _final_body,
        grid=grid,
        in_specs=[
            part,
            rows,
            degp,
            mat,
            pl.BlockSpec((D, D_OUT), lambda i: (0, 0)),
            pl.BlockSpec((1, D_OUT), lambda i: (0, 0)),
        ],
        out_specs=pl.BlockSpec((R, D_OUT), _row_blk),
        out_shape=jax.ShapeDtypeStruct((N, D_OUT), jnp.float32),
    )
    return fc1, layer, final


def kernel(x, edge_index, W1, b1, conv_ws, W2, b2):
    N, D = x.shape
    E = edge_index.shape[1]
    D_OUT = W2.shape[1]
    nlayers = conv_ws.shape[0]

    nchunk = -(-E // (NW * CH))
    ep = NW * nchunk * CH - E
    row3 = jnp.concatenate(
        [edge_index[0], jnp.zeros((ep,), jnp.int32)]
    ).reshape(NW, nchunk, CH)
    col3 = jnp.concatenate(
        [edge_index[1], jnp.full((ep,), N, jnp.int32)]
    ).reshape(NW, nchunk, CH)

    deg_k = _make_deg_kernel(nchunk, N)
    agg_k = _make_agg_kernel(nchunk, N, D)
    fc1, layer, final = _make_tc_kernels(N, D, D_OUT, 2000)

    degp = deg_k(col3).reshape(NC, N).T
    h, g = fc1(x, W1, b1.reshape(1, D), degp)
    x0 = h
    for i in range(nlayers - 1):
        part = agg_k(g, row3, col3)
        g = layer(part, x0, degp, conv_ws[i])
    part = agg_k(g, row3, col3)
    return final(part, x0, degp, conv_ws[nlayers - 1], W2, b2.reshape(1, D_OUT))
